# Initial kernel scaffold; baseline (speedup 1.0000x reference)
import jax
import jax.numpy as jnp
from jax.experimental import pallas as pl


def kernel(x, edge_index, W1, b1, gamma, beta, W2, b2):
    raise NotImplementedError("write your pallas kernel here")


# speedup vs baseline: 8.8416x; 8.8416x over previous
import functools

import jax
import jax.numpy as jnp
from jax import lax
from jax.experimental import pallas as pl
from jax.experimental.pallas import tpu as pltpu
from jax.experimental.pallas import tpu_sc as plsc

N = 10000
E = 320000
D = 128

NC = 2
NS = 16
LANES = 16
NP = 10240
C = 80
ET = E // (NC * NS)
NCH = ET // C
RPS = NP // NS
NBLK = 5
BCH = NCH // NBLK
NB = 2

_MESH = dict(core_axis_name="c", subcore_axis_name="s",
             num_cores=NC, num_subcores=NS)


def _sc_degrees(src_r, dst_r, zeros):
    mesh = plsc.VectorSubcoreMesh(**_MESH)

    @functools.partial(
        pl.kernel,
        out_type=jax.ShapeDtypeStruct((NC, NP, D), jnp.float32),
        mesh=mesh,
        scratch_types=[
            pltpu.VMEM((2, BCH, C), jnp.int32),
            pltpu.VMEM((C, D), jnp.float32),
            pltpu.VMEM((C, D), jnp.float32),
            pltpu.VMEM_SHARED((NP, D), jnp.float32),
        ],
    )
    def k(src_hbm, dst_hbm, z_hbm, out_hbm, idx, ones0, ones1, acc):
        core = lax.axis_index("c")
        sub = lax.axis_index("s")

        @pl.loop(0, C)
        def _(r):
            for cc in range(D // LANES):
                lo = jnp.zeros((LANES,), jnp.float32)
                hi = jnp.ones((LANES,), jnp.float32)
                left = cc < (D // LANES) // 2
                ones0[r, pl.ds(cc * LANES, LANES)] = hi if left else lo
                ones1[r, pl.ds(cc * LANES, LANES)] = lo if left else hi

        pltpu.sync_copy(z_hbm.at[pl.ds(sub * RPS, RPS)],
                        acc.at[pl.ds(sub * RPS, RPS)])
        plsc.subcore_barrier()

        for b in range(NBLK):
            pltpu.sync_copy(src_hbm.at[core, sub, b], idx.at[0])
            pltpu.sync_copy(dst_hbm.at[core, sub, b], idx.at[1])

            @pl.loop(0, BCH)
            def _(j):
                pltpu.sync_copy(ones0, acc.at[idx.at[0].at[j]], add=True)
                pltpu.sync_copy(ones1, acc.at[idx.at[1].at[j]], add=True)

        plsc.subcore_barrier()
        pltpu.sync_copy(acc.at[pl.ds(sub * RPS, RPS)],
                        out_hbm.at[core, pl.ds(sub * RPS, RPS)])

    return k(src_r, dst_r, zeros)


def _sc_edge_pass(h, src_r, dst_r, zeros):
    mesh = plsc.VectorSubcoreMesh(**_MESH)

    @functools.partial(
        pl.kernel,
        out_type=jax.ShapeDtypeStruct((NC, NP, D), jnp.float32),
        mesh=mesh,
        scratch_types=[
            pltpu.VMEM((2, NB, BCH, C), jnp.int32),
            pltpu.VMEM((C, D), jnp.float32),
            pltpu.VMEM((C, D), jnp.float32),
            pltpu.VMEM_SHARED((NP, D), jnp.float32),
            pltpu.SemaphoreType.DMA,
            pltpu.SemaphoreType.DMA,
            pltpu.SemaphoreType.DMA,
        ],
    )
    def k(h_hbm, src_hbm, dst_hbm, z_hbm, out_hbm,
          idx, rows0, rows1, acc, sem0, sem1, semi):
        core = lax.axis_index("c")
        sub = lax.axis_index("s")

        pltpu.sync_copy(src_hbm.at[core, sub, 0], idx.at[0, 0])
        pltpu.sync_copy(dst_hbm.at[core, sub, 0], idx.at[1, 0])
        pltpu.sync_copy(z_hbm.at[pl.ds(sub * RPS, RPS)],
                        acc.at[pl.ds(sub * RPS, RPS)])
        plsc.subcore_barrier()

        for b in range(NBLK):
            sidx = idx.at[0, b % NB]
            didx = idx.at[1, b % NB]
            if b + 1 < NBLK:
                pltpu.async_copy(src_hbm.at[core, sub, b + 1],
                                 idx.at[0, (b + 1) % NB], semi)
                pltpu.async_copy(dst_hbm.at[core, sub, b + 1],
                                 idx.at[1, (b + 1) % NB], semi)

            pltpu.async_copy(h_hbm.at[sidx.at[0]], rows0, sem0)

            @pl.loop(0, (BCH - 1) // 2)
            def _(i):
                j = 2 * i
                pltpu.async_copy(h_hbm.at[sidx.at[j + 1]], rows1, sem1)
                pltpu.make_async_copy(h_hbm.at[sidx.at[j]], rows0,
                                      sem0).wait()
                pltpu.sync_copy(rows0, acc.at[didx.at[j]], add=True)
                pltpu.async_copy(h_hbm.at[sidx.at[j + 2]], rows0, sem0)
                pltpu.make_async_copy(h_hbm.at[sidx.at[j + 1]], rows1,
                                      sem1).wait()
                pltpu.sync_copy(rows1, acc.at[didx.at[j + 1]], add=True)

            pltpu.make_async_copy(h_hbm.at[sidx.at[BCH - 1]], rows0,
                                  sem0).wait()
            pltpu.sync_copy(rows0, acc.at[didx.at[BCH - 1]], add=True)

            if b + 1 < NBLK:
                pltpu.make_async_copy(src_hbm.at[core, sub, b + 1],
                                      idx.at[0, (b + 1) % NB], semi).wait()
                pltpu.make_async_copy(dst_hbm.at[core, sub, b + 1],
                                      idx.at[1, (b + 1) % NB], semi).wait()

        plsc.subcore_barrier()
        pltpu.sync_copy(acc.at[pl.ds(sub * RPS, RPS)],
                        out_hbm.at[core, pl.ds(sub * RPS, RPS)])

    return k(h, src_r, dst_r, zeros)


_R = 2000


def _tc_mm(x, w):
    def body(x_ref, w_ref, o_ref):
        o_ref[...] = jnp.dot(x_ref[...], w_ref[...],
                             preferred_element_type=jnp.float32)

    return pl.pallas_call(
        body,
        grid=(N // _R,),
        in_specs=[pl.BlockSpec((_R, D), lambda i: (i, 0)),
                  pl.BlockSpec((D, D), lambda i: (0, 0))],
        out_specs=pl.BlockSpec((_R, D), lambda i: (i, 0)),
        out_shape=jax.ShapeDtypeStruct((N, D), jnp.float32),
    )(x, w)


def _tc_norm_scale(degp, u):
    def body(dp_ref, u_ref, h_ref, ns_ref, nd_ref):
        dego = dp_ref[0, :, 0:1] + dp_ref[1, :, 0:1]
        degi = dp_ref[0, :, 64:65] + dp_ref[1, :, 64:65]

        def nrm(dg):
            return jnp.where(dg > 0, lax.rsqrt(jnp.maximum(dg, 1.0)), 0.0)

        ns = jnp.broadcast_to(nrm(dego), ns_ref.shape)
        nd = jnp.broadcast_to(nrm(degi), nd_ref.shape)
        ns_ref[...] = ns
        nd_ref[...] = nd
        h_ref[...] = u_ref[...] * ns

    return pl.pallas_call(
        body,
        grid=(N // _R,),
        in_specs=[pl.BlockSpec((NC, _R, D), lambda i: (0, i, 0)),
                  pl.BlockSpec((_R, D), lambda i: (i, 0))],
        out_specs=[pl.BlockSpec((_R, D), lambda i: (i, 0)),
                   pl.BlockSpec((_R, D), lambda i: (i, 0)),
                   pl.BlockSpec((_R, D), lambda i: (i, 0))],
        out_shape=[jax.ShapeDtypeStruct((N, D), jnp.float32),
                   jax.ShapeDtypeStruct((N, D), jnp.float32),
                   jax.ShapeDtypeStruct((N, D), jnp.float32)],
    )(degp, u)


def _tc_comb_stats(p, nd, b1):
    def body(p_ref, nd_ref, b_ref, h_ref, st_ref):
        i = pl.program_id(0)
        h = (p_ref[0] + p_ref[1]) * nd_ref[...] + b_ref[...]
        h_ref[...] = h
        st = jnp.concatenate(
            [jnp.sum(h, axis=0, keepdims=True),
             jnp.sum(h * h, axis=0, keepdims=True)], axis=0)

        @pl.when(i == 0)
        def _():
            st_ref[...] = st

        @pl.when(i != 0)
        def _():
            st_ref[...] += st

    return pl.pallas_call(
        body,
        grid=(N // _R,),
        in_specs=[pl.BlockSpec((NC, _R, D), lambda i: (0, i, 0)),
                  pl.BlockSpec((_R, D), lambda i: (i, 0)),
                  pl.BlockSpec((1, D), lambda i: (0, 0))],
        out_specs=[pl.BlockSpec((_R, D), lambda i: (i, 0)),
                   pl.BlockSpec((2, D), lambda i: (0, 0))],
        out_shape=[jax.ShapeDtypeStruct((N, D), jnp.float32),
                   jax.ShapeDtypeStruct((2, D), jnp.float32)],
    )(p, nd, b1)


def _tc_bn_mm(h, st, gamma, beta, s, w):
    def body(h_ref, st_ref, g_ref, bt_ref, s_ref, w_ref, o_ref):
        mean = st_ref[0:1] / N
        var = st_ref[1:2] / N - mean * mean
        a = g_ref[...] * lax.rsqrt(var + 1e-5)
        c = bt_ref[...] - mean * a
        hh = jnp.maximum(h_ref[...] * a + c, 0.0) * s_ref[...]
        o_ref[...] = jnp.dot(hh, w_ref[...],
                             preferred_element_type=jnp.float32)

    return pl.pallas_call(
        body,
        grid=(N // _R,),
        in_specs=[pl.BlockSpec((_R, D), lambda i: (i, 0)),
                  pl.BlockSpec((2, D), lambda i: (0, 0)),
                  pl.BlockSpec((1, D), lambda i: (0, 0)),
                  pl.BlockSpec((1, D), lambda i: (0, 0)),
                  pl.BlockSpec((_R, D), lambda i: (i, 0)),
                  pl.BlockSpec((D, D), lambda i: (0, 0))],
        out_specs=pl.BlockSpec((_R, D), lambda i: (i, 0)),
        out_shape=jax.ShapeDtypeStruct((N, D), jnp.float32),
    )(h, st, gamma, beta, s, w)


def _tc_final(q, nd, b2):
    def body(q_ref, nd_ref, b_ref, o_ref):
        o_ref[...] = (q_ref[0] + q_ref[1]) * nd_ref[...] + b_ref[...]

    return pl.pallas_call(
        body,
        grid=(N // _R,),
        in_specs=[pl.BlockSpec((NC, _R, D), lambda i: (0, i, 0)),
                  pl.BlockSpec((_R, D), lambda i: (i, 0)),
                  pl.BlockSpec((1, D), lambda i: (0, 0))],
        out_specs=pl.BlockSpec((_R, D), lambda i: (i, 0)),
        out_shape=jax.ShapeDtypeStruct((N, D), jnp.float32),
    )(q, nd, b2)


def kernel(x, edge_index, W1, b1, gamma, beta, W2, b2):
    src_r5 = edge_index[0].reshape(NC, NS, NBLK, BCH, C)
    dst_r5 = edge_index[1].reshape(NC, NS, NBLK, BCH, C)
    zeros = jnp.zeros((NP, D), jnp.float32)

    u = _tc_mm(x, W1)
    degp = _sc_degrees(src_r5, dst_r5, zeros)
    h1, ns, nd = _tc_norm_scale(degp, u)

    p = _sc_edge_pass(h1, src_r5, dst_r5, zeros)
    h, st = _tc_comb_stats(p, nd, b1.reshape(1, D))
    h2 = _tc_bn_mm(h, st, gamma.reshape(1, D), beta.reshape(1, D), ns, W2)
    q = _sc_edge_pass(h2, src_r5, dst_r5, zeros)
    return _tc_final(q, nd, b2.reshape(1, D))


# speedup vs baseline: 10.8187x; 1.2236x over previous
import functools

import jax
import jax.numpy as jnp
from jax import lax
from jax.experimental import pallas as pl
from jax.experimental.pallas import tpu as pltpu
from jax.experimental.pallas import tpu_sc as plsc

N = 10000
E = 320000
D = 128

NC = 2
NS = 16
LANES = 16
NP = 10240
C = 80
ET = E // (NC * NS)
NCH = ET // C
RPS = NP // NS
NBLK = 5
BCH = NCH // NBLK
NB = 2

_MESH = dict(core_axis_name="c", subcore_axis_name="s",
             num_cores=NC, num_subcores=NS)


def _sc_degrees(src_r, dst_r, zeros):
    mesh = plsc.VectorSubcoreMesh(**_MESH)

    @functools.partial(
        pl.kernel,
        out_type=jax.ShapeDtypeStruct((NC, 2, NP), jnp.float32),
        mesh=mesh,
        scratch_types=[
            pltpu.VMEM((2, NB, BCH, C), jnp.int32),
            pltpu.VMEM((C,), jnp.float32),
            pltpu.VMEM_SHARED((NP,), jnp.float32),
            pltpu.VMEM_SHARED((NP,), jnp.float32),
            pltpu.SemaphoreType.DMA,
        ],
    )
    def k(src_hbm, dst_hbm, z_hbm, out_hbm, idx, ones, acc_o, acc_i, semi):
        core = lax.axis_index("c")
        sub = lax.axis_index("s")

        @pl.loop(0, C // LANES)
        def _(r):
            ones[pl.ds(r * LANES, LANES)] = jnp.ones((LANES,), jnp.float32)

        pltpu.sync_copy(src_hbm.at[core, sub, 0], idx.at[0, 0])
        pltpu.sync_copy(dst_hbm.at[core, sub, 0], idx.at[1, 0])
        pltpu.sync_copy(z_hbm.at[pl.ds(sub * RPS, RPS)],
                        acc_o.at[pl.ds(sub * RPS, RPS)])
        pltpu.sync_copy(z_hbm.at[pl.ds(sub * RPS, RPS)],
                        acc_i.at[pl.ds(sub * RPS, RPS)])
        plsc.subcore_barrier()

        for b in range(NBLK):
            sidx = idx.at[0, b % NB]
            didx = idx.at[1, b % NB]
            if b + 1 < NBLK:
                pltpu.async_copy(src_hbm.at[core, sub, b + 1],
                                 idx.at[0, (b + 1) % NB], semi)
                pltpu.async_copy(dst_hbm.at[core, sub, b + 1],
                                 idx.at[1, (b + 1) % NB], semi)

            @pl.loop(0, BCH)
            def _(j):
                pltpu.sync_copy(ones, acc_o.at[sidx.at[j]], add=True)
                pltpu.sync_copy(ones, acc_i.at[didx.at[j]], add=True)

            if b + 1 < NBLK:
                pltpu.make_async_copy(src_hbm.at[core, sub, b + 1],
                                      idx.at[0, (b + 1) % NB], semi).wait()
                pltpu.make_async_copy(dst_hbm.at[core, sub, b + 1],
                                      idx.at[1, (b + 1) % NB], semi).wait()

        plsc.subcore_barrier()
        pltpu.sync_copy(acc_o.at[pl.ds(sub * RPS, RPS)],
                        out_hbm.at[core, 0, pl.ds(sub * RPS, RPS)])
        pltpu.sync_copy(acc_i.at[pl.ds(sub * RPS, RPS)],
                        out_hbm.at[core, 1, pl.ds(sub * RPS, RPS)])

    return k(src_r, dst_r, zeros)


def _sc_edge_pass(h, src_r, dst_r, zeros):
    mesh = plsc.VectorSubcoreMesh(**_MESH)

    @functools.partial(
        pl.kernel,
        out_type=jax.ShapeDtypeStruct((NC, NP, D), jnp.float32),
        mesh=mesh,
        scratch_types=[
            pltpu.VMEM((2, NB, BCH, C), jnp.int32),
            pltpu.VMEM((C, D), jnp.float32),
            pltpu.VMEM((C, D), jnp.float32),
            pltpu.VMEM_SHARED((NP, D), jnp.float32),
            pltpu.SemaphoreType.DMA,
            pltpu.SemaphoreType.DMA,
            pltpu.SemaphoreType.DMA,
        ],
    )
    def k(h_hbm, src_hbm, dst_hbm, z_hbm, out_hbm,
          idx, rows0, rows1, acc, sem0, sem1, semi):
        core = lax.axis_index("c")
        sub = lax.axis_index("s")

        pltpu.sync_copy(src_hbm.at[core, sub, 0], idx.at[0, 0])
        pltpu.sync_copy(dst_hbm.at[core, sub, 0], idx.at[1, 0])
        pltpu.sync_copy(z_hbm.at[pl.ds(sub * RPS, RPS)],
                        acc.at[pl.ds(sub * RPS, RPS)])
        plsc.subcore_barrier()

        for b in range(NBLK):
            sidx = idx.at[0, b % NB]
            didx = idx.at[1, b % NB]
            if b + 1 < NBLK:
                pltpu.async_copy(src_hbm.at[core, sub, b + 1],
                                 idx.at[0, (b + 1) % NB], semi)
                pltpu.async_copy(dst_hbm.at[core, sub, b + 1],
                                 idx.at[1, (b + 1) % NB], semi)

            pltpu.async_copy(h_hbm.at[sidx.at[0]], rows0, sem0)

            @pl.loop(0, (BCH - 1) // 2)
            def _(i):
                j = 2 * i
                pltpu.async_copy(h_hbm.at[sidx.at[j + 1]], rows1, sem1)
                pltpu.make_async_copy(h_hbm.at[sidx.at[j]], rows0,
                                      sem0).wait()
                pltpu.sync_copy(rows0, acc.at[didx.at[j]], add=True)
                pltpu.async_copy(h_hbm.at[sidx.at[j + 2]], rows0, sem0)
                pltpu.make_async_copy(h_hbm.at[sidx.at[j + 1]], rows1,
                                      sem1).wait()
                pltpu.sync_copy(rows1, acc.at[didx.at[j + 1]], add=True)

            pltpu.make_async_copy(h_hbm.at[sidx.at[BCH - 1]], rows0,
                                  sem0).wait()
            pltpu.sync_copy(rows0, acc.at[didx.at[BCH - 1]], add=True)

            if b + 1 < NBLK:
                pltpu.make_async_copy(src_hbm.at[core, sub, b + 1],
                                      idx.at[0, (b + 1) % NB], semi).wait()
                pltpu.make_async_copy(dst_hbm.at[core, sub, b + 1],
                                      idx.at[1, (b + 1) % NB], semi).wait()

        plsc.subcore_barrier()
        pltpu.sync_copy(acc.at[pl.ds(sub * RPS, RPS)],
                        out_hbm.at[core, pl.ds(sub * RPS, RPS)])

    return k(h, src_r, dst_r, zeros)


_R = 2000


def _tc_mm(x, w):
    def body(x_ref, w_ref, o_ref):
        o_ref[...] = jnp.dot(x_ref[...], w_ref[...],
                             preferred_element_type=jnp.float32)

    return pl.pallas_call(
        body,
        grid=(N // _R,),
        in_specs=[pl.BlockSpec((_R, D), lambda i: (i, 0)),
                  pl.BlockSpec((D, D), lambda i: (0, 0))],
        out_specs=pl.BlockSpec((_R, D), lambda i: (i, 0)),
        out_shape=jax.ShapeDtypeStruct((N, D), jnp.float32),
    )(x, w)


def _nrm(dg):
    return jnp.where(dg > 0, lax.rsqrt(jnp.maximum(dg, 1.0)), 0.0)


_DEG_SPEC = pl.BlockSpec((NC, 2, _R, 1), lambda i: (0, 0, i, 0))


def _tc_norm_scale(degp4, u):
    def body(dp_ref, u_ref, h_ref):
        ns = _nrm(dp_ref[0, 0] + dp_ref[1, 0])
        h_ref[...] = u_ref[...] * ns

    return pl.pallas_call(
        body,
        grid=(N // _R,),
        in_specs=[_DEG_SPEC,
                  pl.BlockSpec((_R, D), lambda i: (i, 0))],
        out_specs=pl.BlockSpec((_R, D), lambda i: (i, 0)),
        out_shape=jax.ShapeDtypeStruct((N, D), jnp.float32),
    )(degp4, u)


def _tc_comb_stats(p, degp4, b1):
    def body(p_ref, dp_ref, b_ref, h_ref, st_ref):
        i = pl.program_id(0)
        nd = _nrm(dp_ref[0, 1] + dp_ref[1, 1])
        h = (p_ref[0] + p_ref[1]) * nd + b_ref[...]
        h_ref[...] = h
        st = jnp.concatenate(
            [jnp.sum(h, axis=0, keepdims=True),
             jnp.sum(h * h, axis=0, keepdims=True)], axis=0)

        @pl.when(i == 0)
        def _():
            st_ref[...] = st

        @pl.when(i != 0)
        def _():
            st_ref[...] += st

    return pl.pallas_call(
        body,
        grid=(N // _R,),
        in_specs=[pl.BlockSpec((NC, _R, D), lambda i: (0, i, 0)),
                  _DEG_SPEC,
                  pl.BlockSpec((1, D), lambda i: (0, 0))],
        out_specs=[pl.BlockSpec((_R, D), lambda i: (i, 0)),
                   pl.BlockSpec((2, D), lambda i: (0, 0))],
        out_shape=[jax.ShapeDtypeStruct((N, D), jnp.float32),
                   jax.ShapeDtypeStruct((2, D), jnp.float32)],
    )(p, degp4, b1)


def _tc_bn_mm(h, st, gamma, beta, degp4, w):
    def body(h_ref, st_ref, g_ref, bt_ref, dp_ref, w_ref, o_ref):
        mean = st_ref[0:1] / N
        var = st_ref[1:2] / N - mean * mean
        a = g_ref[...] * lax.rsqrt(var + 1e-5)
        c = bt_ref[...] - mean * a
        ns = _nrm(dp_ref[0, 0] + dp_ref[1, 0])
        hh = jnp.maximum(h_ref[...] * a + c, 0.0) * ns
        o_ref[...] = jnp.dot(hh, w_ref[...],
                             preferred_element_type=jnp.float32)

    return pl.pallas_call(
        body,
        grid=(N // _R,),
        in_specs=[pl.BlockSpec((_R, D), lambda i: (i, 0)),
                  pl.BlockSpec((2, D), lambda i: (0, 0)),
                  pl.BlockSpec((1, D), lambda i: (0, 0)),
                  pl.BlockSpec((1, D), lambda i: (0, 0)),
                  _DEG_SPEC,
                  pl.BlockSpec((D, D), lambda i: (0, 0))],
        out_specs=pl.BlockSpec((_R, D), lambda i: (i, 0)),
        out_shape=jax.ShapeDtypeStruct((N, D), jnp.float32),
    )(h, st, gamma, beta, degp4, w)


def _tc_final(q, degp4, b2):
    def body(q_ref, dp_ref, b_ref, o_ref):
        nd = _nrm(dp_ref[0, 1] + dp_ref[1, 1])
        o_ref[...] = (q_ref[0] + q_ref[1]) * nd + b_ref[...]

    return pl.pallas_call(
        body,
        grid=(N // _R,),
        in_specs=[pl.BlockSpec((NC, _R, D), lambda i: (0, i, 0)),
                  _DEG_SPEC,
                  pl.BlockSpec((1, D), lambda i: (0, 0))],
        out_specs=pl.BlockSpec((_R, D), lambda i: (i, 0)),
        out_shape=jax.ShapeDtypeStruct((N, D), jnp.float32),
    )(q, degp4, b2)


def kernel(x, edge_index, W1, b1, gamma, beta, W2, b2):
    src_r5 = edge_index[0].reshape(NC, NS, NBLK, BCH, C)
    dst_r5 = edge_index[1].reshape(NC, NS, NBLK, BCH, C)
    zeros = jnp.zeros((NP, D), jnp.float32)
    zeros1 = jnp.zeros((NP,), jnp.float32)

    u = _tc_mm(x, W1)
    degp = _sc_degrees(src_r5, dst_r5, zeros1)
    degp4 = degp.reshape(NC, 2, NP, 1)
    h1 = _tc_norm_scale(degp4, u)

    p = _sc_edge_pass(h1, src_r5, dst_r5, zeros)
    h, st = _tc_comb_stats(p, degp4, b1.reshape(1, D))
    h2 = _tc_bn_mm(h, st, gamma.reshape(1, D), beta.reshape(1, D), degp4, W2)
    q = _sc_edge_pass(h2, src_r5, dst_r5, zeros)
    return _tc_final(q, degp4, b2.reshape(1, D))
